# Initial kernel scaffold; baseline (speedup 1.0000x reference)
import jax
import jax.numpy as jnp
from jax.experimental import pallas as pl


def kernel(x, edge_index, edge_weight, W1, b1, W2, b2):
    raise NotImplementedError("write your pallas kernel here")


# speedup vs baseline: 4.2685x; 4.2685x over previous
import functools

import jax
import jax.numpy as jnp
from jax import lax
from jax.experimental import pallas as pl
from jax.experimental.pallas import tpu as pltpu
from jax.experimental.pallas import tpu_sc as plsc

N_NODES = 10000
D_IN = 128
HIDDEN = 128
LABEL_DIM = 40
LABEL_PAD = 48

NC = 2
NS = 16
K = 128


def _make_sc_spmm(n_pad, d, e_pad):
    n_workers = NC * NS
    per_w = e_pad // n_workers
    n_chunks = per_w // K
    rows_per_tile = n_pad // NS
    full, rem = divmod(rows_per_tile, K)
    mesh = plsc.VectorSubcoreMesh(core_axis_name="c", subcore_axis_name="s")

    @functools.partial(
        pl.kernel,
        out_type=jax.ShapeDtypeStruct((NC, n_pad, d), jnp.float32),
        mesh=mesh,
        scratch_types=[
            pltpu.VMEM((K,), jnp.int32),
            pltpu.VMEM((K,), jnp.int32),
            pltpu.VMEM((K,), jnp.float32),
            pltpu.VMEM((K, d), jnp.float32),
            pltpu.VMEM_SHARED((n_pad, d), jnp.float32),
            pltpu.SemaphoreType.DMA,
        ],
        compiler_params=pltpu.CompilerParams(use_tc_tiling_on_sc=False),
    )
    def spmm(src_hbm, dst_hbm, w_hbm, h_hbm, out_hbm,
             src_v, dst_v, w_v, rows_v, acc_sh, sem):
        c = lax.axis_index("c")
        s = lax.axis_index("s")
        wid = s * NC + c
        ebase = wid * per_w
        row0 = s * rows_per_tile

        def zrow(i, carry):
            for j in range(d // 16):
                rows_v[i, pl.ds(j * 16, 16)] = jnp.zeros((16,), jnp.float32)
            return carry
        lax.fori_loop(0, K, zrow, 0)
        for q in range(full):
            pltpu.sync_copy(rows_v, acc_sh.at[pl.ds(row0 + q * K, K)])
        if rem:
            pltpu.sync_copy(rows_v.at[pl.ds(0, rem)],
                            acc_sh.at[pl.ds(row0 + full * K, rem)])
        plsc.subcore_barrier()

        def chunk(k, carry):
            eoff = ebase + k * K
            pltpu.sync_copy(src_hbm.at[pl.ds(eoff, K)], src_v)
            pltpu.sync_copy(dst_hbm.at[pl.ds(eoff, K)], dst_v)
            pltpu.sync_copy(w_hbm.at[pl.ds(eoff, K)], w_v)
            pltpu.async_copy(h_hbm.at[src_v], rows_v, sem).wait()

            def grp(g, inner):
                w16 = w_v[pl.ds(g * 16, 16)]
                for lane in range(16):
                    row = g * 16 + lane
                    wb = lax.gather(
                        w16, jnp.full((16, 1), lane, jnp.int32),
                        lax.GatherDimensionNumbers(
                            offset_dims=(), collapsed_slice_dims=(0,),
                            start_index_map=(0,)),
                        (1,), mode=lax.GatherScatterMode.PROMISE_IN_BOUNDS)
                    for j in range(d // 16):
                        sl = pl.ds(j * 16, 16)
                        rows_v[row, sl] = rows_v[row, sl] * wb
                return inner
            lax.fori_loop(0, K // 16, grp, 0)
            pltpu.sync_copy(rows_v, acc_sh.at[dst_v], add=True)
            return carry
        lax.fori_loop(0, n_chunks, chunk, 0)
        plsc.subcore_barrier()

        pltpu.sync_copy(acc_sh.at[pl.ds(row0, rows_per_tile)],
                        out_hbm.at[c, pl.ds(row0, rows_per_tile)])

    return spmm


def _mm_body(x_ref, w_ref, o_ref):
    o_ref[...] = jnp.dot(x_ref[...], w_ref[...],
                         preferred_element_type=jnp.float32)


def _fuse1_body(p_ref, b_ref, w_ref, o_ref):
    h = jnp.maximum(p_ref[0] + p_ref[1] + b_ref[...], 0.0)
    o_ref[...] = jnp.dot(h, w_ref[...], preferred_element_type=jnp.float32)


def _fuse2_body(p_ref, b_ref, o_ref):
    s = p_ref[0] + p_ref[1] + b_ref[...]
    logits = s[:, :LABEL_DIM]
    m = jnp.max(logits, axis=1, keepdims=True)
    z = logits - m
    lse = jnp.log(jnp.sum(jnp.exp(z), axis=1, keepdims=True))
    o_ref[...] = z - lse


def kernel(x, edge_index, edge_weight, W1, b1, W2, b2):
    n = x.shape[0]
    e = edge_weight.shape[0]
    chunk_span = NC * NS * K
    e_pad = ((e + chunk_span - 1) // chunk_span) * chunk_span
    row_span = NS * 8
    n_pad = ((n + row_span - 1) // row_span) * row_span

    src = edge_index[0].astype(jnp.int32)
    dst = edge_index[1].astype(jnp.int32)
    pad = e_pad - e
    if pad:
        src = jnp.pad(src, (0, pad))
        dst = jnp.pad(dst, (0, pad))
        edge_weight = jnp.pad(edge_weight, (0, pad))

    w2p = jnp.pad(W2, ((0, 0), (0, LABEL_PAD - LABEL_DIM)))
    b1r = b1.reshape(1, HIDDEN)
    b2r = jnp.pad(b2, (0, LABEL_PAD - LABEL_DIM)).reshape(1, LABEL_PAD)

    h1 = pl.pallas_call(
        _mm_body,
        out_shape=jax.ShapeDtypeStruct((n, HIDDEN), jnp.float32),
    )(x, W1)

    spmm1 = _make_sc_spmm(n_pad, HIDDEN, e_pad)
    p1 = spmm1(src, dst, edge_weight, h1)

    h2 = pl.pallas_call(
        _fuse1_body,
        out_shape=jax.ShapeDtypeStruct((n_pad, LABEL_PAD), jnp.float32),
    )(p1, b1r, w2p)

    spmm2 = _make_sc_spmm(n_pad, LABEL_PAD, e_pad)
    p2 = spmm2(src, dst, edge_weight, h2)

    out = pl.pallas_call(
        _fuse2_body,
        out_shape=jax.ShapeDtypeStruct((n_pad, LABEL_DIM), jnp.float32),
    )(p2, b2r)
    return out[:n]


# speedup vs baseline: 4.5396x; 1.0635x over previous
import functools

import jax
import jax.numpy as jnp
from jax import lax
from jax.experimental import pallas as pl
from jax.experimental.pallas import tpu as pltpu
from jax.experimental.pallas import tpu_sc as plsc

N_NODES = 10000
D_IN = 128
HIDDEN = 128
LABEL_DIM = 40
LABEL_PAD = 48

NC = 2
NS = 16
K = 128


def _broadcast_lane(v16, lane):
    return lax.gather(
        v16, jnp.full((16, 1), lane, jnp.int32),
        lax.GatherDimensionNumbers(
            offset_dims=(), collapsed_slice_dims=(0,), start_index_map=(0,)),
        (1,), mode=lax.GatherScatterMode.PROMISE_IN_BOUNDS)


def _make_sc_spmm(n_pad, d, e_pad):
    n_workers = NC * NS
    per_w = e_pad // n_workers
    n_chunks = per_w // K
    assert n_chunks % 2 == 0
    rows_per_tile = n_pad // NS
    full, rem = divmod(rows_per_tile, K)
    mesh = plsc.VectorSubcoreMesh(core_axis_name="c", subcore_axis_name="s")

    @functools.partial(
        pl.kernel,
        out_type=jax.ShapeDtypeStruct((NC, n_pad, d), jnp.float32),
        mesh=mesh,
        scratch_types=[
            pltpu.VMEM((2, 2, K), jnp.int32),
            pltpu.VMEM((2, K), jnp.float32),
            pltpu.VMEM((2, K, d), jnp.float32),
            pltpu.VMEM_SHARED((n_pad, d), jnp.float32),
            pltpu.SemaphoreType.DMA,
            pltpu.SemaphoreType.DMA,
            pltpu.SemaphoreType.DMA,
            pltpu.SemaphoreType.DMA,
        ],
        compiler_params=pltpu.CompilerParams(use_tc_tiling_on_sc=False),
    )
    def spmm(pk_hbm, w_hbm, h_hbm, out_hbm, pk_v, w_v, rows_v, acc_sh,
             gsem0, gsem1, ssem0, ssem1):
        c = lax.axis_index("c")
        s = lax.axis_index("s")
        wid = s * NC + c
        cbase = wid * n_chunks
        row0 = s * rows_per_tile
        gsem = (gsem0, gsem1)
        ssem = (ssem0, ssem1)

        def rows_b(b):
            return rows_v.at[b]

        def gather_start(b):
            pltpu.async_copy(h_hbm.at[pk_v.at[b, 0]], rows_b(b), gsem[b])

        def gather_wait(b):
            pltpu.make_async_copy(
                h_hbm.at[pk_v.at[b, 0]], rows_b(b), gsem[b]).wait()

        def scatter_start(b):
            pltpu.async_copy(
                rows_b(b), acc_sh.at[pk_v.at[b, 1]], ssem[b], add=True)

        def scatter_wait(b):
            pltpu.make_async_copy(
                rows_b(b), acc_sh.at[pk_v.at[b, 1]], ssem[b]).wait()

        def zrow(i, carry):
            for j in range(d // 16):
                rows_v[0, i, pl.ds(j * 16, 16)] = jnp.zeros((16,), jnp.float32)
            return carry
        lax.fori_loop(0, K, zrow, 0)
        for q in range(full):
            pltpu.sync_copy(rows_v.at[0], acc_sh.at[pl.ds(row0 + q * K, K)])
        if rem:
            pltpu.sync_copy(rows_v.at[0, pl.ds(0, rem)],
                            acc_sh.at[pl.ds(row0 + full * K, rem)])
        plsc.subcore_barrier()

        def scale(b):
            def grp(g, inner):
                w16 = w_v[b, pl.ds(g * 16, 16)]
                for lane in range(16):
                    row = g * 16 + lane
                    wb = _broadcast_lane(w16, lane)
                    for j in range(d // 16):
                        sl = pl.ds(j * 16, 16)
                        rows_v[b, row, sl] = rows_v[b, row, sl] * wb
                return inner
            lax.fori_loop(0, K // 16, grp, 0)

        pltpu.sync_copy(pk_hbm.at[cbase], pk_v.at[0])
        pltpu.sync_copy(w_hbm.at[cbase], w_v.at[0])
        gather_start(0)

        def half(k, b):
            nb = 1 - b
            @pl.when(k >= 1)
            def _():
                scatter_wait(nb)

            @pl.when(k + 1 < n_chunks)
            def _():
                pltpu.sync_copy(pk_hbm.at[cbase + k + 1], pk_v.at[nb])
                pltpu.sync_copy(w_hbm.at[cbase + k + 1], w_v.at[nb])
                gather_start(nb)

            gather_wait(b)
            scale(b)
            scatter_start(b)

        def pair(k2, carry):
            half(k2 * 2, 0)
            half(k2 * 2 + 1, 1)
            return carry
        lax.fori_loop(0, n_chunks // 2, pair, 0)
        scatter_wait(1)
        plsc.subcore_barrier()

        pltpu.sync_copy(acc_sh.at[pl.ds(row0, rows_per_tile)],
                        out_hbm.at[c, pl.ds(row0, rows_per_tile)])

    return spmm


def _mm_body(x_ref, w_ref, o_ref):
    o_ref[...] = jnp.dot(x_ref[...], w_ref[...],
                         preferred_element_type=jnp.float32)


def _fuse1_body(p_ref, b_ref, w_ref, o_ref):
    h = jnp.maximum(p_ref[0] + p_ref[1] + b_ref[...], 0.0)
    o_ref[...] = jnp.dot(h, w_ref[...], preferred_element_type=jnp.float32)


def _fuse2_body(p_ref, b_ref, o_ref):
    s = p_ref[0] + p_ref[1] + b_ref[...]
    logits = s[:, :LABEL_DIM]
    m = jnp.max(logits, axis=1, keepdims=True)
    z = logits - m
    lse = jnp.log(jnp.sum(jnp.exp(z), axis=1, keepdims=True))
    o_ref[...] = z - lse


def kernel(x, edge_index, edge_weight, W1, b1, W2, b2):
    n = x.shape[0]
    e = edge_weight.shape[0]
    chunk_span = NC * NS * K * 2
    e_pad = ((e + chunk_span - 1) // chunk_span) * chunk_span
    row_span = NS * 8
    n_pad = ((n + row_span - 1) // row_span) * row_span

    src = edge_index[0].astype(jnp.int32)
    dst = edge_index[1].astype(jnp.int32)
    pad = e_pad - e
    if pad:
        src = jnp.pad(src, (0, pad))
        dst = jnp.pad(dst, (0, pad))
        edge_weight = jnp.pad(edge_weight, (0, pad))
    packed = jnp.stack([src.reshape(-1, K), dst.reshape(-1, K)], axis=1)
    wchunk = edge_weight.reshape(-1, K)

    w2p = jnp.pad(W2, ((0, 0), (0, LABEL_PAD - LABEL_DIM)))
    b1r = b1.reshape(1, HIDDEN)
    b2r = jnp.pad(b2, (0, LABEL_PAD - LABEL_DIM)).reshape(1, LABEL_PAD)

    h1 = pl.pallas_call(
        _mm_body,
        out_shape=jax.ShapeDtypeStruct((n, HIDDEN), jnp.float32),
    )(x, W1)

    spmm1 = _make_sc_spmm(n_pad, HIDDEN, e_pad)
    p1 = spmm1(packed, wchunk, h1)

    h2 = pl.pallas_call(
        _fuse1_body,
        out_shape=jax.ShapeDtypeStruct((n_pad, LABEL_PAD), jnp.float32),
    )(p1, b1r, w2p)

    spmm2 = _make_sc_spmm(n_pad, LABEL_PAD, e_pad)
    p2 = spmm2(packed, wchunk, h2)

    out = pl.pallas_call(
        _fuse2_body,
        out_shape=jax.ShapeDtypeStruct((n_pad, LABEL_DIM), jnp.float32),
    )(p2, b2r)
    return out[:n]
